# baseline (speedup 1.0000x reference)
import functools

import jax
import jax.numpy as jnp
from jax import lax
from jax.experimental import pallas as pl
from jax.experimental.pallas import tpu as pltpu
from jax.experimental.pallas import tpu_sc as plsc

_WEIGHT = 0.0

_B = 16384
_D = 128
_NW = 32
_RPW = _B // _NW
_CH = 128
_CHD = _CH * _D
_NCHUNK = _RPW // _CH


def _sqrt16(x):
    i = plsc.bitcast(x, jnp.int32)
    y = plsc.bitcast((i >> 1) + jnp.int32(0x1FBD1DF5), jnp.float32)
    y = 0.5 * (y + x / y)
    y = 0.5 * (y + x / y)
    y = 0.5 * (y + x / y)
    return y


def _sc_body(h_hbm, r_hbm, t_hbm, out_hbm, h_v, r_v, t_v, out_v, sems):
    wid = lax.axis_index("s") * 2 + lax.axis_index("c")
    row0 = wid * _RPW
    rowstride = jnp.arange(16, dtype=jnp.int32) * _D

    def dma_triplet(c, buf):
        elem0 = (row0 + c * _CH) * _D
        dst = pl.ds(buf * _CHD, _CHD)
        copies = (
            pltpu.make_async_copy(h_hbm.at[pl.ds(elem0, _CHD)], h_v.at[dst],
                                  sems.at[buf, 0]),
            pltpu.make_async_copy(r_hbm.at[pl.ds(elem0, _CHD)], r_v.at[dst],
                                  sems.at[buf, 1]),
            pltpu.make_async_copy(t_hbm.at[pl.ds(elem0, _CHD)], t_v.at[dst],
                                  sems.at[buf, 2]),
        )
        return copies

    def start(c, buf):
        for cp in dma_triplet(c, buf):
            cp.start()

    def wait(c, buf):
        for cp in dma_triplet(c, buf):
            cp.wait()

    start(0, 0)

    def chunk_body(c, _):
        buf = lax.rem(c, 2)

        @pl.when(c + 1 < _NCHUNK)
        def _():
            start(c + 1, 1 - buf)

        wait(c, buf)
        vbase = buf * _CHD

        def group_body(g, _):
            goff = vbase + g * (16 * _D)
            acc = jnp.zeros((16,), jnp.float32)
            for col in range(_D):
                idx = rowstride + (goff + col)
                hv = plsc.load_gather(h_v, [idx])
                rv = plsc.load_gather(r_v, [idx])
                tv = plsc.load_gather(t_v, [idx])
                d = hv + rv - tv
                acc = acc + d * d
            out_v[pl.ds(c * _CH + g * 16, 16)] = _WEIGHT * _sqrt16(acc)
            return 0

        lax.fori_loop(0, _CH // 16, group_body, 0)
        return 0

    lax.fori_loop(0, _NCHUNK, chunk_body, 0)
    pltpu.sync_copy(out_v, out_hbm.at[pl.ds(row0, _RPW)])


def kernel(h_emb, r_emb, t_emb):
    mesh = plsc.VectorSubcoreMesh(core_axis_name="c", subcore_axis_name="s")
    k = functools.partial(
        pl.kernel,
        mesh=mesh,
        compiler_params=pltpu.CompilerParams(needs_layout_passes=False),
        out_type=jax.ShapeDtypeStruct((_B,), jnp.float32),
        scratch_types=[
            pltpu.VMEM((2 * _CHD,), jnp.float32),
            pltpu.VMEM((2 * _CHD,), jnp.float32),
            pltpu.VMEM((2 * _CHD,), jnp.float32),
            pltpu.VMEM((_RPW,), jnp.float32),
            pltpu.SemaphoreType.DMA((2, 3)),
        ],
    )(_sc_body)
    return k(h_emb.reshape(-1), r_emb.reshape(-1), t_emb.reshape(-1))
